# Initial kernel scaffold; baseline (speedup 1.0000x reference)
import jax
import jax.numpy as jnp
from jax.experimental import pallas as pl


def kernel(node_states, edges, kernel, kernel_attention):
    raise NotImplementedError("write your pallas kernel here")


# speedup vs baseline: 8.5933x; 8.5933x over previous
import functools

import jax
import jax.numpy as jnp
from jax import lax
from jax.experimental import pallas as pl
from jax.experimental.pallas import tpu as pltpu
from jax.experimental.pallas import tpu_sc as plsc

NC = 2
NS = 16
L = 16
NW = NC * NS
C = 128
AW = 144


def _tc_transform(node_states, w, a2):
    n, d = node_states.shape
    u = w.shape[1]
    bn = 512
    grid = (pl.cdiv(n, bn),)

    def body(ns_ref, w_ref, a2_ref, h_ref, sdss_ref):
        h = jnp.dot(ns_ref[...], w_ref[...], preferred_element_type=jnp.float32)
        h_ref[...] = h
        sdss_ref[...] = jnp.dot(h, a2_ref[...], preferred_element_type=jnp.float32)

    return pl.pallas_call(
        body,
        grid=grid,
        in_specs=[
            pl.BlockSpec((bn, d), lambda i: (i, 0)),
            pl.BlockSpec((d, u), lambda i: (0, 0)),
            pl.BlockSpec((u, 2), lambda i: (0, 0)),
        ],
        out_specs=[
            pl.BlockSpec((bn, u), lambda i: (i, 0)),
            pl.BlockSpec((bn, 2), lambda i: (i, 0)),
        ],
        out_shape=[
            jax.ShapeDtypeStruct((n, u), jnp.float32),
            jax.ShapeDtypeStruct((n, 2), jnp.float32),
        ],
    )(node_states, w, a2)


def _sc_edge_pass(dst_p, src_p, sd, ss, h, n_pad, n_edges_real):
    e_pad = dst_p.shape[0]
    epw = e_pad // NW
    n_chunks = epw // C
    rpt = n_pad // NS
    r_full = rpt // C
    r_rem = rpt - r_full * C
    u = h.shape[1]
    nvec = u // L

    mesh = plsc.VectorSubcoreMesh(core_axis_name="c", subcore_axis_name="s")

    @functools.partial(
        pl.kernel,
        out_type=jax.ShapeDtypeStruct((NC, n_pad, AW), jnp.float32),
        mesh=mesh,
        scratch_types=[
            pltpu.VMEM_SHARED((n_pad, AW), jnp.float32),
            pltpu.VMEM((C,), jnp.int32),
            pltpu.VMEM((C,), jnp.int32),
            pltpu.VMEM((C,), jnp.float32),
            pltpu.VMEM((C,), jnp.float32),
            pltpu.VMEM((C,), jnp.float32),
            pltpu.VMEM((C, u), jnp.float32),
            pltpu.VMEM((C, AW), jnp.float32),
            pltpu.SemaphoreType.DMA,
            pltpu.SemaphoreType.DMA,
            pltpu.SemaphoreType.DMA,
        ],
        compiler_params=pltpu.CompilerParams(use_tc_tiling_on_sc=False,
                                             needs_layout_passes=False),
    )
    def k(dst_hbm, src_hbm, sd_hbm, ss_hbm, h_hbm, out_hbm,
          accum, dsti, srci, sdv, ssv, wbuf, hrows, scaled,
          sem1, sem2, sem3):
        cid = lax.axis_index("c")
        sid = lax.axis_index("s")
        wid = cid * NS + sid

        @pl.loop(0, C)
        def _(r):
            for kk in range(AW // L):
                scaled[r, pl.ds(kk * L, L)] = jnp.zeros((L,), jnp.float32)

        rbase = sid * rpt
        for p in range(r_full):
            pltpu.sync_copy(scaled, accum.at[pl.ds(rbase + p * C, C)])
        if r_rem:
            pltpu.sync_copy(scaled.at[pl.ds(0, r_rem)],
                            accum.at[pl.ds(rbase + r_full * C, r_rem)])
        plsc.subcore_barrier()

        ebase = wid * epw

        @pl.loop(0, n_chunks)
        def _(i):
            base = ebase + i * C
            pltpu.sync_copy(dst_hbm.at[pl.ds(base, C)], dsti)
            pltpu.sync_copy(src_hbm.at[pl.ds(base, C)], srci)
            cp1 = pltpu.async_copy(sd_hbm.at[dsti], sdv, sem1)
            cp2 = pltpu.async_copy(ss_hbm.at[srci], ssv, sem2)
            cp3 = pltpu.async_copy(h_hbm.at[srci], hrows, sem3)
            cp1.wait()
            cp2.wait()

            for j in range(C // L):
                sl = pl.ds(j * L, L)
                s = sdv[sl] + ssv[sl]
                s = jnp.maximum(s, s * jnp.float32(0.2))
                s = jnp.minimum(jnp.maximum(s, jnp.float32(-2.0)),
                                jnp.float32(2.0))
                wv = jnp.exp(s)
                gid = base + j * L + lax.iota(jnp.int32, L)
                wv = jnp.where(gid < n_edges_real, wv, jnp.float32(0.0))
                wbuf[sl] = wv

            cp3.wait()

            @pl.loop(0, C)
            def _(e2):
                wspl = plsc.load_gather(
                    wbuf, [jnp.full((L,), e2, dtype=jnp.int32)])
                for kk in range(nvec):
                    sl = pl.ds(kk * L, L)
                    scaled[e2, sl] = hrows[e2, sl] * wspl
                scaled[e2, pl.ds(u, L)] = wspl

            pltpu.sync_copy(scaled, accum.at[dsti], add=True)

        plsc.subcore_barrier()

        for p in range(r_full):
            pltpu.sync_copy(accum.at[pl.ds(rbase + p * C, C)],
                            out_hbm.at[cid, pl.ds(rbase + p * C, C)])
        if r_rem:
            pltpu.sync_copy(accum.at[pl.ds(rbase + r_full * C, r_rem)],
                            out_hbm.at[cid, pl.ds(rbase + r_full * C, r_rem)])

    return k(dst_p, src_p, sd, ss, h)


def _tc_combine(acc, n, u):
    bn = 512
    grid = (pl.cdiv(n, bn),)

    def body(acc_ref, out_ref):
        a = acc_ref[0]
        b = acc_ref[1]
        num = a[:, :u] + b[:, :u]
        den = a[:, u:u + 1] + b[:, u:u + 1]
        out_ref[...] = jnp.where(den > jnp.float32(0.0), num / den,
                                 jnp.float32(0.0))

    return pl.pallas_call(
        body,
        grid=grid,
        in_specs=[pl.BlockSpec((NC, bn, AW), lambda i: (0, i, 0))],
        out_specs=pl.BlockSpec((bn, u), lambda i: (i, 0)),
        out_shape=jax.ShapeDtypeStruct((n, u), jnp.float32),
    )(acc)


def kernel(node_states, edges, kernel, kernel_attention):
    n, d = node_states.shape
    u = kernel.shape[1]
    e = edges.shape[0]

    e32 = edges.astype(jnp.int32)
    dst = e32[:, 0]
    src = e32[:, 1]
    e_pad = ((e + NW * C - 1) // (NW * C)) * (NW * C)
    if e_pad != e:
        pad = jnp.zeros((e_pad - e,), jnp.int32)
        dst = jnp.concatenate([dst, pad])
        src = jnp.concatenate([src, pad])

    a2 = kernel_attention.reshape(2, u).T

    h, sdss = _tc_transform(node_states, kernel, a2)
    sd = sdss[:, 0]
    ss = sdss[:, 1]

    n_pad = ((n + NS * 8 - 1) // (NS * 8)) * (NS * 8)
    acc = _sc_edge_pass(dst, src, sd, ss, h, n_pad, e)
    out = _tc_combine(acc, n, u)
    return out


# speedup vs baseline: 10.5969x; 1.2332x over previous
import functools

import jax
import jax.numpy as jnp
from jax import lax
from jax.experimental import pallas as pl
from jax.experimental.pallas import tpu as pltpu
from jax.experimental.pallas import tpu_sc as plsc

NC = 2
NS = 16
L = 16
NW = NC * NS
C = 96
NBUF = 3
NIDS = 6


def _tc_transform(node_states, w, a2):
    n, d = node_states.shape
    u = w.shape[1]
    bn = 512
    grid = (pl.cdiv(n, bn),)

    def body(ns_ref, w_ref, a2_ref, h_ref, sdss_ref):
        h = jnp.dot(ns_ref[...], w_ref[...], preferred_element_type=jnp.float32)
        h_ref[...] = h
        sdss_ref[...] = jnp.dot(h, a2_ref[...], preferred_element_type=jnp.float32)

    return pl.pallas_call(
        body,
        grid=grid,
        in_specs=[
            pl.BlockSpec((bn, d), lambda i: (i, 0)),
            pl.BlockSpec((d, u), lambda i: (0, 0)),
            pl.BlockSpec((u, 2), lambda i: (0, 0)),
        ],
        out_specs=[
            pl.BlockSpec((bn, u), lambda i: (i, 0)),
            pl.BlockSpec((bn, 2), lambda i: (i, 0)),
        ],
        out_shape=[
            jax.ShapeDtypeStruct((n, u), jnp.float32),
            jax.ShapeDtypeStruct((n, 2), jnp.float32),
        ],
    )(node_states, w, a2)


def _sc_edge_pass(dst_p, src_p, sd, ss, h, n_pad, n_edges_real):
    e_pad = dst_p.shape[0]
    epw = e_pad // NW
    n_chunks = epw // C
    rpt = n_pad // NS
    r_full = rpt // C
    r_rem = rpt - r_full * C
    u = h.shape[1]
    nvec = u // L

    dst2 = dst_p.reshape(NW * n_chunks, C)
    src2 = src_p.reshape(NW * n_chunks, C)

    mesh = plsc.VectorSubcoreMesh(core_axis_name="c", subcore_axis_name="s")

    @functools.partial(
        pl.kernel,
        out_type=(
            jax.ShapeDtypeStruct((NC, n_pad, u), jnp.float32),
            jax.ShapeDtypeStruct((NW, n_pad), jnp.float32),
        ),
        mesh=mesh,
        scratch_types=[
            pltpu.VMEM_SHARED((n_pad, u), jnp.float32),
            pltpu.VMEM((n_pad,), jnp.float32),
            pltpu.VMEM((NIDS, C), jnp.int32),
            pltpu.VMEM((NIDS, C), jnp.int32),
            pltpu.VMEM((NBUF, C), jnp.float32),
            pltpu.VMEM((NBUF, C), jnp.float32),
            pltpu.VMEM((C,), jnp.float32),
            pltpu.VMEM((NBUF, C, u), jnp.float32),
        ] + [pltpu.SemaphoreType.DMA] * (2 * NBUF + NIDS),
        compiler_params=pltpu.CompilerParams(use_tc_tiling_on_sc=False,
                                             needs_layout_passes=False),
    )
    def k(dst_hbm, src_hbm, sd_hbm, ss_hbm, h_hbm, out_hbm, den_hbm,
          accum, den_local, dsti, srci, sdv, ssv, wbuf, hrows, *sems):
        cid = lax.axis_index("c")
        sid = lax.axis_index("s")
        wid = cid * NS + sid
        sem_g = sems[0:NBUF]
        sem_s = sems[NBUF:2 * NBUF]
        sem_id = sems[2 * NBUF:]
        row0 = wid * n_chunks

        def issue_ids(ch, i6):
            pltpu.async_copy(dst_hbm.at[row0 + ch], dsti.at[i6], sem_id[i6])
            pltpu.async_copy(src_hbm.at[row0 + ch], srci.at[i6], sem_id[i6])

        def wait_ids(ch, i6):
            pltpu.make_async_copy(dst_hbm.at[row0 + ch], dsti.at[i6],
                                  sem_id[i6]).wait()
            pltpu.make_async_copy(src_hbm.at[row0 + ch], srci.at[i6],
                                  sem_id[i6]).wait()

        def issue_gathers(i6, b):
            pltpu.async_copy(sd_hbm.at[dsti.at[i6]], sdv.at[b], sem_g[b])
            pltpu.async_copy(ss_hbm.at[srci.at[i6]], ssv.at[b], sem_g[b])
            pltpu.async_copy(h_hbm.at[srci.at[i6]], hrows.at[b], sem_g[b])

        def wait_gathers(i6, b):
            pltpu.make_async_copy(sd_hbm.at[dsti.at[i6]], sdv.at[b],
                                  sem_g[b]).wait()
            pltpu.make_async_copy(ss_hbm.at[srci.at[i6]], ssv.at[b],
                                  sem_g[b]).wait()
            pltpu.make_async_copy(h_hbm.at[srci.at[i6]], hrows.at[b],
                                  sem_g[b]).wait()

        def wait_scatter(i6, b):
            pltpu.make_async_copy(hrows.at[b], accum.at[dsti.at[i6]],
                                  sem_s[b]).wait()

        pltpu.sync_copy(dst_hbm.at[row0], dsti.at[0])
        pltpu.sync_copy(src_hbm.at[row0], srci.at[0])
        issue_ids(1, 1)
        issue_gathers(0, 0)

        @pl.loop(0, C)
        def _(r):
            for kk in range(nvec):
                hrows[NBUF - 1, r, pl.ds(kk * L, L)] = jnp.zeros(
                    (L,), jnp.float32)

        @pl.loop(0, n_pad // L)
        def _(r):
            den_local[pl.ds(r * L, L)] = jnp.zeros((L,), jnp.float32)

        rbase = sid * rpt
        for p in range(r_full):
            pltpu.sync_copy(hrows.at[NBUF - 1],
                            accum.at[pl.ds(rbase + p * C, C)])
        if r_rem:
            pltpu.sync_copy(hrows.at[NBUF - 1, pl.ds(0, r_rem)],
                            accum.at[pl.ds(rbase + r_full * C, r_rem)])
        plsc.subcore_barrier()

        ebase = wid * epw

        @pl.loop(0, n_chunks, step=NIDS)
        def _(i0):
            for b6 in range(NIDS):
                ch = i0 + b6
                b3 = b6 % NBUF
                q3 = (b6 + 1) % NBUF
                q6 = (b6 + 1) % NIDS
                r6 = (b6 + 2) % NIDS

                @pl.when(ch >= 2)
                def _():
                    wait_scatter((b6 + 4) % NIDS, q3)

                @pl.when(ch + 2 < n_chunks)
                def _():
                    issue_ids(ch + 2, r6)

                @pl.when(ch + 1 < n_chunks)
                def _():
                    wait_ids(ch + 1, q6)
                    issue_gathers(q6, q3)

                wait_gathers(b6, b3)

                base = ebase + ch * C
                for j in range(C // L):
                    sl = pl.ds(j * L, L)
                    s = sdv[b3, sl] + ssv[b3, sl]
                    s = jnp.maximum(s, s * jnp.float32(0.2))
                    s = jnp.minimum(jnp.maximum(s, jnp.float32(-2.0)),
                                    jnp.float32(2.0))
                    wv = jnp.exp(s)
                    gid = base + j * L + lax.iota(jnp.int32, L)
                    wv = jnp.where(gid < n_edges_real, wv, jnp.float32(0.0))
                    wbuf[sl] = wv
                    plsc.addupdate_scatter(den_local, [dsti[b6, sl]], wv)

                @pl.loop(0, C)
                def _(e2):
                    wspl = plsc.load_gather(
                        wbuf, [jnp.full((L,), e2, dtype=jnp.int32)])
                    for kk in range(nvec):
                        sl = pl.ds(kk * L, L)
                        hrows[b3, e2, sl] = hrows[b3, e2, sl] * wspl

                pltpu.async_copy(hrows.at[b3], accum.at[dsti.at[b6]],
                                 sem_s[b3], add=True)

        for ch in range(max(0, n_chunks - 2), n_chunks):
            wait_scatter(ch % NIDS, ch % NBUF)

        plsc.subcore_barrier()

        for p in range(r_full):
            pltpu.sync_copy(accum.at[pl.ds(rbase + p * C, C)],
                            out_hbm.at[cid, pl.ds(rbase + p * C, C)])
        if r_rem:
            pltpu.sync_copy(accum.at[pl.ds(rbase + r_full * C, r_rem)],
                            out_hbm.at[cid, pl.ds(rbase + r_full * C, r_rem)])
        pltpu.sync_copy(den_local, den_hbm.at[wid])

    return k(dst2, src2, sd, ss, h)


def _tc_combine(acc, den_parts, n, u):
    bn = 512
    grid = (pl.cdiv(n, bn),)

    def body(acc_ref, den_ref, out_ref):
        num = acc_ref[0] + acc_ref[1]
        den = jnp.sum(den_ref[...], axis=0)[:, None]
        out_ref[...] = jnp.where(den > jnp.float32(0.0), num / den,
                                 jnp.float32(0.0))

    return pl.pallas_call(
        body,
        grid=grid,
        in_specs=[
            pl.BlockSpec((NC, bn, u), lambda i: (0, i, 0)),
            pl.BlockSpec((NW, bn), lambda i: (0, i)),
        ],
        out_specs=pl.BlockSpec((bn, u), lambda i: (i, 0)),
        out_shape=jax.ShapeDtypeStruct((n, u), jnp.float32),
    )(acc, den_parts)


def kernel(node_states, edges, kernel, kernel_attention):
    n, d = node_states.shape
    u = kernel.shape[1]
    e = edges.shape[0]

    e32 = edges.astype(jnp.int32)
    dst = e32[:, 0]
    src = e32[:, 1]
    egrain = NIDS * NW * C
    e_pad = ((e + egrain - 1) // egrain) * egrain
    if e_pad != e:
        pad = jnp.zeros((e_pad - e,), jnp.int32)
        dst = jnp.concatenate([dst, pad])
        src = jnp.concatenate([src, pad])

    a2 = kernel_attention.reshape(2, u).T

    h, sdss = _tc_transform(node_states, kernel, a2)
    sd = sdss[:, 0]
    ss = sdss[:, 1]

    n_pad = ((n + NS * 8 - 1) // (NS * 8)) * (NS * 8)
    acc, den_parts = _sc_edge_pass(dst, src, sd, ss, h, n_pad, e)
    out = _tc_combine(acc, den_parts, n, u)
    return out


# speedup vs baseline: 10.9056x; 1.0291x over previous
import functools

import jax
import jax.numpy as jnp
from jax import lax
from jax.experimental import pallas as pl
from jax.experimental.pallas import tpu as pltpu
from jax.experimental.pallas import tpu_sc as plsc

NC = 2
NS = 16
L = 16
NW = NC * NS
C = 96
NBUF = 3
NIDS = 6


def _tc_transform(node_states, w, a2):
    n, d = node_states.shape
    u = w.shape[1]
    bn = 512
    grid = (pl.cdiv(n, bn),)

    def body(ns_ref, w_ref, a2_ref, h_ref, sdss_ref):
        h = jnp.dot(ns_ref[...], w_ref[...], preferred_element_type=jnp.float32)
        h_ref[...] = h
        sdss_ref[...] = jnp.dot(h, a2_ref[...], preferred_element_type=jnp.float32)

    return pl.pallas_call(
        body,
        grid=grid,
        in_specs=[
            pl.BlockSpec((bn, d), lambda i: (i, 0)),
            pl.BlockSpec((d, u), lambda i: (0, 0)),
            pl.BlockSpec((u, 2), lambda i: (0, 0)),
        ],
        out_specs=[
            pl.BlockSpec((bn, u), lambda i: (i, 0)),
            pl.BlockSpec((bn, 2), lambda i: (i, 0)),
        ],
        out_shape=[
            jax.ShapeDtypeStruct((n, u), jnp.float32),
            jax.ShapeDtypeStruct((n, 2), jnp.float32),
        ],
    )(node_states, w, a2)


def _sc_edge_pass(dst_p, src_p, sd, ss, h, n_pad, n_edges_real):
    e_pad = dst_p.shape[0]
    epw = e_pad // NW
    n_chunks = epw // C
    rpt = n_pad // NS
    r_full = rpt // C
    r_rem = rpt - r_full * C
    u = h.shape[1]
    nvec = u // L

    dst2 = dst_p.reshape(NW * n_chunks, C)
    src2 = src_p.reshape(NW * n_chunks, C)

    mesh = plsc.VectorSubcoreMesh(core_axis_name="c", subcore_axis_name="s")

    @functools.partial(
        pl.kernel,
        out_type=(
            jax.ShapeDtypeStruct((NC, n_pad, u), jnp.float32),
            jax.ShapeDtypeStruct((NW, n_pad), jnp.float32),
        ),
        mesh=mesh,
        scratch_types=[
            pltpu.VMEM_SHARED((n_pad, u), jnp.float32),
            pltpu.VMEM((n_pad,), jnp.float32),
            pltpu.VMEM((NIDS, C), jnp.int32),
            pltpu.VMEM((NIDS, C), jnp.int32),
            pltpu.VMEM((NBUF, C), jnp.float32),
            pltpu.VMEM((NBUF, C), jnp.float32),
            pltpu.VMEM((C,), jnp.float32),
            pltpu.VMEM((NBUF, C, u), jnp.float32),
        ] + [pltpu.SemaphoreType.DMA] * (2 * NBUF + NIDS),
        compiler_params=pltpu.CompilerParams(use_tc_tiling_on_sc=False,
                                             needs_layout_passes=False),
    )
    def k(dst_hbm, src_hbm, sd_hbm, ss_hbm, h_hbm, out_hbm, den_hbm,
          accum, den_local, dsti, srci, sdv, ssv, wbuf, hrows, *sems):
        cid = lax.axis_index("c")
        sid = lax.axis_index("s")
        wid = cid * NS + sid
        sem_g = sems[0:NBUF]
        sem_s = sems[NBUF:2 * NBUF]
        sem_id = sems[2 * NBUF:]
        row0 = wid * n_chunks

        def issue_ids(ch, i6):
            pltpu.async_copy(dst_hbm.at[row0 + ch], dsti.at[i6], sem_id[i6])
            pltpu.async_copy(src_hbm.at[row0 + ch], srci.at[i6], sem_id[i6])

        def wait_ids(ch, i6):
            pltpu.make_async_copy(dst_hbm.at[row0 + ch], dsti.at[i6],
                                  sem_id[i6]).wait()
            pltpu.make_async_copy(src_hbm.at[row0 + ch], srci.at[i6],
                                  sem_id[i6]).wait()

        def issue_gathers(i6, b):
            pltpu.async_copy(sd_hbm.at[dsti.at[i6]], sdv.at[b], sem_g[b])
            pltpu.async_copy(ss_hbm.at[srci.at[i6]], ssv.at[b], sem_g[b])
            pltpu.async_copy(h_hbm.at[srci.at[i6]], hrows.at[b], sem_g[b])

        def wait_gathers(i6, b):
            pltpu.make_async_copy(sd_hbm.at[dsti.at[i6]], sdv.at[b],
                                  sem_g[b]).wait()
            pltpu.make_async_copy(ss_hbm.at[srci.at[i6]], ssv.at[b],
                                  sem_g[b]).wait()
            pltpu.make_async_copy(h_hbm.at[srci.at[i6]], hrows.at[b],
                                  sem_g[b]).wait()

        def wait_scatter(i6, b):
            pltpu.make_async_copy(hrows.at[b], accum.at[dsti.at[i6]],
                                  sem_s[b]).wait()

        pltpu.sync_copy(dst_hbm.at[row0], dsti.at[0])
        pltpu.sync_copy(src_hbm.at[row0], srci.at[0])
        issue_ids(1, 1)
        issue_gathers(0, 0)

        @pl.loop(0, C)
        def _(r):
            for kk in range(nvec):
                hrows[NBUF - 1, r, pl.ds(kk * L, L)] = jnp.zeros(
                    (L,), jnp.float32)

        @pl.loop(0, n_pad // L)
        def _(r):
            den_local[pl.ds(r * L, L)] = jnp.zeros((L,), jnp.float32)

        rbase = sid * rpt
        for p in range(r_full):
            pltpu.sync_copy(hrows.at[NBUF - 1],
                            accum.at[pl.ds(rbase + p * C, C)])
        if r_rem:
            pltpu.sync_copy(hrows.at[NBUF - 1, pl.ds(0, r_rem)],
                            accum.at[pl.ds(rbase + r_full * C, r_rem)])
        plsc.subcore_barrier()

        ebase = wid * epw

        @pl.loop(0, n_chunks, step=NIDS)
        def _(i0):
            for b6 in range(NIDS):
                ch = i0 + b6
                b3 = b6 % NBUF
                q3 = (b6 + 1) % NBUF
                q6 = (b6 + 1) % NIDS
                r6 = (b6 + 2) % NIDS

                @pl.when(ch >= 2)
                def _():
                    wait_scatter((b6 + 4) % NIDS, q3)

                @pl.when(ch + 2 < n_chunks)
                def _():
                    issue_ids(ch + 2, r6)

                @pl.when(ch + 1 < n_chunks)
                def _():
                    wait_ids(ch + 1, q6)
                    issue_gathers(q6, q3)

                wait_gathers(b6, b3)

                base = ebase + ch * C
                for j in range(C // L):
                    sl = pl.ds(j * L, L)
                    s = sdv[b3, sl] + ssv[b3, sl]
                    s = jnp.maximum(s, s * jnp.float32(0.2))
                    s = jnp.minimum(jnp.maximum(s, jnp.float32(-2.0)),
                                    jnp.float32(2.0))
                    wv = jnp.exp(s)
                    gid = base + j * L + lax.iota(jnp.int32, L)
                    wv = jnp.where(gid < n_edges_real, wv, jnp.float32(0.0))
                    wbuf[sl] = wv
                    plsc.addupdate_scatter(den_local, [dsti[b6, sl]], wv)

                @plsc.parallel_loop(0, C, unroll=8)
                def _(e2):
                    wspl = plsc.load_gather(
                        wbuf, [jnp.full((L,), e2, dtype=jnp.int32)])
                    for kk in range(nvec):
                        sl = pl.ds(kk * L, L)
                        hrows[b3, e2, sl] = hrows[b3, e2, sl] * wspl

                pltpu.async_copy(hrows.at[b3], accum.at[dsti.at[b6]],
                                 sem_s[b3], add=True)

        for ch in range(max(0, n_chunks - 2), n_chunks):
            wait_scatter(ch % NIDS, ch % NBUF)

        plsc.subcore_barrier()

        for p in range(r_full):
            pltpu.sync_copy(accum.at[pl.ds(rbase + p * C, C)],
                            out_hbm.at[cid, pl.ds(rbase + p * C, C)])
        if r_rem:
            pltpu.sync_copy(accum.at[pl.ds(rbase + r_full * C, r_rem)],
                            out_hbm.at[cid, pl.ds(rbase + r_full * C, r_rem)])
        pltpu.sync_copy(den_local, den_hbm.at[wid])

    return k(dst2, src2, sd, ss, h)


def _tc_combine(acc, den_parts, n, u):
    bn = 512
    grid = (pl.cdiv(n, bn),)

    def body(acc_ref, den_ref, out_ref):
        num = acc_ref[0] + acc_ref[1]
        den = jnp.sum(den_ref[...], axis=0)[:, None]
        out_ref[...] = jnp.where(den > jnp.float32(0.0), num / den,
                                 jnp.float32(0.0))

    return pl.pallas_call(
        body,
        grid=grid,
        in_specs=[
            pl.BlockSpec((NC, bn, u), lambda i: (0, i, 0)),
            pl.BlockSpec((NW, bn), lambda i: (0, i)),
        ],
        out_specs=pl.BlockSpec((bn, u), lambda i: (i, 0)),
        out_shape=jax.ShapeDtypeStruct((n, u), jnp.float32),
    )(acc, den_parts)


def kernel(node_states, edges, kernel, kernel_attention):
    n, d = node_states.shape
    u = kernel.shape[1]
    e = edges.shape[0]

    e32 = edges.astype(jnp.int32)
    dst = e32[:, 0]
    src = e32[:, 1]
    egrain = NIDS * NW * C
    e_pad = ((e + egrain - 1) // egrain) * egrain
    if e_pad != e:
        pad = jnp.zeros((e_pad - e,), jnp.int32)
        dst = jnp.concatenate([dst, pad])
        src = jnp.concatenate([src, pad])

    a2 = kernel_attention.reshape(2, u).T

    h, sdss = _tc_transform(node_states, kernel, a2)
    sd = sdss[:, 0]
    ss = sdss[:, 1]

    n_pad = ((n + NS * 8 - 1) // (NS * 8)) * (NS * 8)
    acc, den_parts = _sc_edge_pass(dst, src, sd, ss, h, n_pad, e)
    out = _tc_combine(acc, den_parts, n, u)
    return out
